# Initial kernel scaffold; baseline (speedup 1.0000x reference)
import jax
import jax.numpy as jnp
from jax.experimental import pallas as pl
from jax.experimental.pallas import tpu as pltpu

_NUM_TOKENS = 32768
_D_MODEL = 768
_NUM_EXPERTS = 64
_TOP_K = 2
_BLOCK = 512


def _gate_body(x_ref, w_ref, b_ref, gw_ref, sel_ref, aux_ref, cnt_ref, prob_ref):
    i = pl.program_id(0)
    nsteps = pl.num_programs(0)

    logits = jnp.dot(x_ref[...], w_ref[...], preferred_element_type=jnp.float32)
    logits = logits + b_ref[...]

    iota = jax.lax.broadcasted_iota(jnp.int32, logits.shape, 1)
    m1 = jnp.max(logits, axis=1, keepdims=True)
    i1 = jnp.min(jnp.where(logits == m1, iota, _NUM_EXPERTS), axis=1, keepdims=True)
    sel1 = iota == i1

    masked = jnp.where(sel1, -jnp.inf, logits)
    m2 = jnp.max(masked, axis=1, keepdims=True)
    i2 = jnp.min(jnp.where(masked == m2, iota, _NUM_EXPERTS), axis=1, keepdims=True)
    sel2 = iota == i2

    w1 = 1.0 / (1.0 + jnp.exp(m2 - m1))
    w2 = 1.0 - w1

    gw_ref[...] = jnp.where(sel1, w1, jnp.where(sel2, w2, 0.0))
    sel_ref[...] = jnp.concatenate([i1, i2], axis=1)

    p = jnp.exp(logits - m1)
    p = p / jnp.sum(p, axis=1, keepdims=True)

    @pl.when(i == 0)
    def _():
        cnt_ref[...] = jnp.zeros_like(cnt_ref)
        prob_ref[...] = jnp.zeros_like(prob_ref)

    onehot2 = jnp.where(sel1 | sel2, 1.0, 0.0)
    cnt_ref[...] += jnp.sum(onehot2, axis=0, keepdims=True)
    prob_ref[...] += jnp.sum(p, axis=0, keepdims=True)

    @pl.when(i == nsteps - 1)
    def _():
        scale = _NUM_EXPERTS / (_NUM_TOKENS * _TOP_K * _NUM_TOKENS)
        aux_ref[0, 0] = scale * jnp.sum(cnt_ref[...] * prob_ref[...])


def kernel(x, W, b):
    n, d = x.shape
    e = W.shape[1]
    grid = (n // _BLOCK,)
    gw, sel, aux = pl.pallas_call(
        _gate_body,
        grid=grid,
        in_specs=[
            pl.BlockSpec((_BLOCK, d), lambda i: (i, 0)),
            pl.BlockSpec((d, e), lambda i: (0, 0)),
            pl.BlockSpec((1, e), lambda i: (0, 0)),
        ],
        out_specs=[
            pl.BlockSpec((_BLOCK, e), lambda i: (i, 0)),
            pl.BlockSpec((_BLOCK, _TOP_K), lambda i: (i, 0)),
            pl.BlockSpec((1, 1), lambda i: (0, 0)),
        ],
        out_shape=[
            jax.ShapeDtypeStruct((n, e), jnp.float32),
            jax.ShapeDtypeStruct((n, _TOP_K), jnp.int32),
            jax.ShapeDtypeStruct((1, 1), jnp.float32),
        ],
        scratch_shapes=[
            pltpu.VMEM((1, e), jnp.float32),
            pltpu.VMEM((1, e), jnp.float32),
        ],
    )(x, W, b.reshape(1, e))
    return gw, sel, aux[0, 0]


# speedup vs baseline: 6.2706x; 6.2706x over previous
import jax
import jax.numpy as jnp
from jax.experimental import pallas as pl
from jax.experimental.pallas import tpu as pltpu

_NUM_TOKENS = 32768
_D_MODEL = 768
_NUM_EXPERTS = 64
_TOP_K = 2
_BLOCK = 512


def _gate_body(x_ref, w_ref, b_ref, gw_ref, sel_ref, aux_ref, cnt_ref, prob_ref):
    i = pl.program_id(0)
    nsteps = pl.num_programs(0)

    logits = jnp.dot(x_ref[...], w_ref[...], preferred_element_type=jnp.float32)
    logits = logits + b_ref[...]

    iota = jax.lax.broadcasted_iota(jnp.int32, logits.shape, 1)
    m1 = jnp.max(logits, axis=1, keepdims=True)
    i1 = jnp.min(jnp.where(logits == m1, iota, _NUM_EXPERTS), axis=1, keepdims=True)
    sel1 = iota == i1

    masked = jnp.where(sel1, -jnp.inf, logits)
    m2 = jnp.max(masked, axis=1, keepdims=True)
    i2 = jnp.min(jnp.where(masked == m2, iota, _NUM_EXPERTS), axis=1, keepdims=True)
    sel2 = iota == i2

    w1 = 1.0 / (1.0 + jnp.exp(m2 - m1))
    w2 = 1.0 - w1

    gw_ref[...] = jnp.where(sel1, w1, jnp.where(sel2, w2, 0.0))
    sel_ref[...] = jnp.concatenate([i1, i2], axis=1)

    p = jnp.exp(logits - m1)
    p = p / jnp.sum(p, axis=1, keepdims=True)

    @pl.when(i == 0)
    def _():
        cnt_ref[...] = jnp.zeros_like(cnt_ref)
        prob_ref[...] = jnp.zeros_like(prob_ref)

    onehot2 = jnp.where(sel1 | sel2, 1.0, 0.0)
    cnt_ref[...] += jnp.sum(onehot2, axis=0, keepdims=True)
    prob_ref[...] += jnp.sum(p, axis=0, keepdims=True)

    @pl.when(i == nsteps - 1)
    def _():
        scale = _NUM_EXPERTS / (_NUM_TOKENS * _TOP_K * _NUM_TOKENS)
        aux_ref[...] = scale * jnp.sum(cnt_ref[...] * prob_ref[...], keepdims=True)


def kernel(x, W, b):
    n, d = x.shape
    e = W.shape[1]
    grid = (n // _BLOCK,)
    gw, sel, aux = pl.pallas_call(
        _gate_body,
        grid=grid,
        in_specs=[
            pl.BlockSpec((_BLOCK, d), lambda i: (i, 0)),
            pl.BlockSpec((d, e), lambda i: (0, 0)),
            pl.BlockSpec((1, e), lambda i: (0, 0)),
        ],
        out_specs=[
            pl.BlockSpec((_BLOCK, e), lambda i: (i, 0)),
            pl.BlockSpec((_BLOCK, _TOP_K), lambda i: (i, 0)),
            pl.BlockSpec((1, 1), lambda i: (0, 0)),
        ],
        out_shape=[
            jax.ShapeDtypeStruct((n, e), jnp.float32),
            jax.ShapeDtypeStruct((n, _TOP_K), jnp.int32),
            jax.ShapeDtypeStruct((1, 1), jnp.float32),
        ],
        scratch_shapes=[
            pltpu.VMEM((1, e), jnp.float32),
            pltpu.VMEM((1, e), jnp.float32),
        ],
    )(x, W, b.reshape(1, e))
    return gw, sel, aux[0, 0]


# speedup vs baseline: 7.9036x; 1.2604x over previous
import jax
import jax.numpy as jnp
from jax.experimental import pallas as pl
from jax.experimental.pallas import tpu as pltpu

_NUM_TOKENS = 32768
_D_MODEL = 768
_NUM_EXPERTS = 64
_TOP_K = 2
_BLOCK = 1024


def _gate_body(x_ref, w_ref, b_ref, gw_ref, sel_ref, aux_ref, cnt_ref, prob_ref):
    i = pl.program_id(0)
    nsteps = pl.num_programs(0)

    logits = jnp.dot(x_ref[...], w_ref[...], preferred_element_type=jnp.float32)
    logits = logits + b_ref[...]

    iota = jax.lax.broadcasted_iota(jnp.int32, logits.shape, 1)
    m1 = jnp.max(logits, axis=1, keepdims=True)
    i1 = jnp.min(jnp.where(logits == m1, iota, _NUM_EXPERTS), axis=1, keepdims=True)
    sel1 = iota == i1

    masked = jnp.where(sel1, -jnp.inf, logits)
    m2 = jnp.max(masked, axis=1, keepdims=True)
    i2 = jnp.min(jnp.where(masked == m2, iota, _NUM_EXPERTS), axis=1, keepdims=True)
    sel2 = iota == i2

    w1 = 1.0 / (1.0 + jnp.exp(m2 - m1))
    w2 = 1.0 - w1

    gw_ref[...] = jnp.where(sel1, w1, jnp.where(sel2, w2, 0.0))
    sel_ref[...] = jnp.concatenate([i1, i2], axis=1)

    p = jnp.exp(logits - m1)
    p = p / jnp.sum(p, axis=1, keepdims=True)

    @pl.when(i == 0)
    def _():
        cnt_ref[...] = jnp.zeros_like(cnt_ref)
        prob_ref[...] = jnp.zeros_like(prob_ref)

    onehot2 = jnp.where(sel1 | sel2, 1.0, 0.0)
    cnt_ref[...] += jnp.sum(onehot2, axis=0, keepdims=True)
    prob_ref[...] += jnp.sum(p, axis=0, keepdims=True)

    @pl.when(i == nsteps - 1)
    def _():
        scale = _NUM_EXPERTS / (_NUM_TOKENS * _TOP_K * _NUM_TOKENS)
        aux_ref[...] = scale * jnp.sum(cnt_ref[...] * prob_ref[...], keepdims=True)


def kernel(x, W, b):
    n, d = x.shape
    e = W.shape[1]
    grid = (n // _BLOCK,)
    gw, sel, aux = pl.pallas_call(
        _gate_body,
        grid=grid,
        in_specs=[
            pl.BlockSpec((_BLOCK, d), lambda i: (i, 0)),
            pl.BlockSpec((d, e), lambda i: (0, 0)),
            pl.BlockSpec((1, e), lambda i: (0, 0)),
        ],
        out_specs=[
            pl.BlockSpec((_BLOCK, e), lambda i: (i, 0)),
            pl.BlockSpec((_BLOCK, _TOP_K), lambda i: (i, 0)),
            pl.BlockSpec((1, 1), lambda i: (0, 0)),
        ],
        out_shape=[
            jax.ShapeDtypeStruct((n, e), jnp.float32),
            jax.ShapeDtypeStruct((n, _TOP_K), jnp.int32),
            jax.ShapeDtypeStruct((1, 1), jnp.float32),
        ],
        scratch_shapes=[
            pltpu.VMEM((1, e), jnp.float32),
            pltpu.VMEM((1, e), jnp.float32),
        ],
    )(x, W, b.reshape(1, e))
    return gw, sel, aux[0, 0]


# speedup vs baseline: 7.9377x; 1.0043x over previous
import jax
import jax.numpy as jnp
from jax.experimental import pallas as pl
from jax.experimental.pallas import tpu as pltpu

_NUM_TOKENS = 32768
_D_MODEL = 768
_NUM_EXPERTS = 64
_TOP_K = 2
_BLOCK = 1024


def _gate_body(x_ref, w_ref, b_ref, gw_ref, sel_ref, aux_ref, cnt_ref, prob_ref):
    i = pl.program_id(0)
    nsteps = pl.num_programs(0)

    logits = jnp.dot(x_ref[...], w_ref[...], preferred_element_type=jnp.float32)
    logits = logits + b_ref[...]

    ib = jax.lax.bitcast_convert_type(logits, jnp.int32)
    s = jnp.where(ib < 0, ib ^ jnp.int32(0x7FFFFFFF), ib)
    iota = jax.lax.broadcasted_iota(jnp.int32, logits.shape, 1)
    key = (s & jnp.int32(-64)) | (jnp.int32(_NUM_EXPERTS - 1) - iota)

    k1 = jnp.max(key, axis=1, keepdims=True)
    sel1 = key == k1
    i1 = jnp.int32(_NUM_EXPERTS - 1) - (k1 & jnp.int32(63))

    masked = jnp.where(sel1, jnp.int32(-(2**31)), key)
    k2 = jnp.max(masked, axis=1, keepdims=True)
    sel2 = key == k2
    i2 = jnp.int32(_NUM_EXPERTS - 1) - (k2 & jnp.int32(63))

    sb1 = jnp.where(k1 < 0, k1 ^ jnp.int32(0x7FFFFFFF), k1)
    m1a = jax.lax.bitcast_convert_type(sb1, jnp.float32)

    p_un = jnp.exp(logits - m1a)
    mask2f = jnp.where(sel1 | sel2, 1.0, 0.0).astype(jnp.float32)
    p_sel = p_un * mask2f

    ones_e = jnp.full((_NUM_EXPERTS, _NUM_EXPERTS), 1.0, dtype=jnp.float32)
    s_full = jnp.dot(p_un, ones_e, preferred_element_type=jnp.float32)
    s_pair = jnp.dot(p_sel, ones_e, preferred_element_type=jnp.float32)

    gw_ref[...] = p_sel / s_pair
    sel_ref[...] = jnp.concatenate([i1, i2], axis=1)

    p = p_un / s_full

    @pl.when(i == 0)
    def _():
        cnt_ref[...] = jnp.zeros_like(cnt_ref)
        prob_ref[...] = jnp.zeros_like(prob_ref)

    cnt_ref[...] += jnp.sum(mask2f, axis=0, keepdims=True)
    prob_ref[...] += jnp.sum(p, axis=0, keepdims=True)

    @pl.when(i == nsteps - 1)
    def _():
        scale = _NUM_EXPERTS / (_NUM_TOKENS * _TOP_K * _NUM_TOKENS)
        aux_ref[...] = scale * jnp.sum(cnt_ref[...] * prob_ref[...], keepdims=True)


def kernel(x, W, b):
    n, d = x.shape
    e = W.shape[1]
    grid = (n // _BLOCK,)
    gw, sel, aux = pl.pallas_call(
        _gate_body,
        grid=grid,
        in_specs=[
            pl.BlockSpec((_BLOCK, d), lambda i: (i, 0)),
            pl.BlockSpec((d, e), lambda i: (0, 0)),
            pl.BlockSpec((1, e), lambda i: (0, 0)),
        ],
        out_specs=[
            pl.BlockSpec((_BLOCK, e), lambda i: (i, 0)),
            pl.BlockSpec((_BLOCK, _TOP_K), lambda i: (i, 0)),
            pl.BlockSpec((1, 1), lambda i: (0, 0)),
        ],
        out_shape=[
            jax.ShapeDtypeStruct((n, e), jnp.float32),
            jax.ShapeDtypeStruct((n, _TOP_K), jnp.int32),
            jax.ShapeDtypeStruct((1, 1), jnp.float32),
        ],
        scratch_shapes=[
            pltpu.VMEM((1, e), jnp.float32),
            pltpu.VMEM((1, e), jnp.float32),
        ],
    )(x, W, b.reshape(1, e))
    return gw, sel, aux[0, 0]


# speedup vs baseline: 8.7671x; 1.1045x over previous
import jax
import jax.numpy as jnp
from jax.experimental import pallas as pl
from jax.experimental.pallas import tpu as pltpu

_NUM_TOKENS = 32768
_D_MODEL = 768
_NUM_EXPERTS = 64
_TOP_K = 2
_BLOCK = 2048


def _gate_body(x_ref, w_ref, b_ref, gw_ref, sel_ref, aux_ref, cnt_ref, prob_ref):
    i = pl.program_id(0)
    nsteps = pl.num_programs(0)

    logits = jnp.dot(x_ref[...], w_ref[...], preferred_element_type=jnp.float32)
    logits = logits + b_ref[...]

    iota = jax.lax.broadcasted_iota(jnp.int32, logits.shape, 1)
    m1 = jnp.max(logits, axis=1, keepdims=True)
    i1 = jnp.min(jnp.where(logits == m1, iota, _NUM_EXPERTS), axis=1, keepdims=True)
    sel1 = iota == i1

    masked = jnp.where(sel1, -jnp.inf, logits)
    m2 = jnp.max(masked, axis=1, keepdims=True)
    i2 = jnp.min(jnp.where(masked == m2, iota, _NUM_EXPERTS), axis=1, keepdims=True)
    sel2 = iota == i2

    p_un = jnp.exp(logits - m1)
    mask2f = jnp.where(sel1 | sel2, 1.0, 0.0).astype(jnp.float32)
    p_sel = p_un * mask2f

    ones_e = jnp.full((_NUM_EXPERTS, _NUM_EXPERTS), 1.0, dtype=jnp.float32)
    s_full = jnp.dot(p_un, ones_e, preferred_element_type=jnp.float32)
    s_pair = jnp.dot(p_sel, ones_e, preferred_element_type=jnp.float32)

    gw_ref[...] = p_sel / s_pair
    sel_ref[...] = jnp.concatenate([i1, i2], axis=1)

    p = p_un / s_full

    @pl.when(i == 0)
    def _():
        cnt_ref[...] = jnp.zeros_like(cnt_ref)
        prob_ref[...] = jnp.zeros_like(prob_ref)

    cnt_ref[...] += jnp.sum(mask2f, axis=0, keepdims=True)
    prob_ref[...] += jnp.sum(p, axis=0, keepdims=True)

    @pl.when(i == nsteps - 1)
    def _():
        scale = _NUM_EXPERTS / (_NUM_TOKENS * _TOP_K * _NUM_TOKENS)
        aux_ref[...] = scale * jnp.sum(cnt_ref[...] * prob_ref[...], keepdims=True)


def kernel(x, W, b):
    n, d = x.shape
    e = W.shape[1]
    grid = (n // _BLOCK,)
    gw, sel, aux = pl.pallas_call(
        _gate_body,
        grid=grid,
        in_specs=[
            pl.BlockSpec((_BLOCK, d), lambda i: (i, 0)),
            pl.BlockSpec((d, e), lambda i: (0, 0)),
            pl.BlockSpec((1, e), lambda i: (0, 0)),
        ],
        out_specs=[
            pl.BlockSpec((_BLOCK, e), lambda i: (i, 0)),
            pl.BlockSpec((_BLOCK, _TOP_K), lambda i: (i, 0)),
            pl.BlockSpec((1, 1), lambda i: (0, 0)),
        ],
        out_shape=[
            jax.ShapeDtypeStruct((n, e), jnp.float32),
            jax.ShapeDtypeStruct((n, _TOP_K), jnp.int32),
            jax.ShapeDtypeStruct((1, 1), jnp.float32),
        ],
        scratch_shapes=[
            pltpu.VMEM((1, e), jnp.float32),
            pltpu.VMEM((1, e), jnp.float32),
        ],
    )(x, W, b.reshape(1, e))
    return gw, sel, aux[0, 0]


# speedup vs baseline: 9.2267x; 1.0524x over previous
import jax
import jax.numpy as jnp
from jax.experimental import pallas as pl
from jax.experimental.pallas import tpu as pltpu

_NUM_TOKENS = 32768
_D_MODEL = 768
_NUM_EXPERTS = 64
_TOP_K = 2
_BLOCK = 4096


def _gate_body(x_ref, w_ref, b_ref, gw_ref, sel_ref, aux_ref, cnt_ref, prob_ref):
    i = pl.program_id(0)
    nsteps = pl.num_programs(0)

    logits = jnp.dot(x_ref[...], w_ref[...], preferred_element_type=jnp.float32)
    logits = logits + b_ref[...]

    iota = jax.lax.broadcasted_iota(jnp.int32, logits.shape, 1)
    m1 = jnp.max(logits, axis=1, keepdims=True)
    i1 = jnp.min(jnp.where(logits == m1, iota, _NUM_EXPERTS), axis=1, keepdims=True)
    sel1 = iota == i1

    masked = jnp.where(sel1, -jnp.inf, logits)
    m2 = jnp.max(masked, axis=1, keepdims=True)
    i2 = jnp.min(jnp.where(masked == m2, iota, _NUM_EXPERTS), axis=1, keepdims=True)
    sel2 = iota == i2

    p_un = jnp.exp(logits - m1)
    mask2f = jnp.where(sel1 | sel2, 1.0, 0.0).astype(jnp.float32)
    p_sel = p_un * mask2f

    ones_e = jnp.full((_NUM_EXPERTS, _NUM_EXPERTS), 1.0, dtype=jnp.float32)
    s_full = jnp.dot(p_un, ones_e, preferred_element_type=jnp.float32)
    s_pair = jnp.dot(p_sel, ones_e, preferred_element_type=jnp.float32)

    gw_ref[...] = p_sel / s_pair
    sel_ref[...] = jnp.concatenate([i1, i2], axis=1)

    p = p_un / s_full

    @pl.when(i == 0)
    def _():
        cnt_ref[...] = jnp.zeros_like(cnt_ref)
        prob_ref[...] = jnp.zeros_like(prob_ref)

    cnt_ref[...] += jnp.sum(mask2f, axis=0, keepdims=True)
    prob_ref[...] += jnp.sum(p, axis=0, keepdims=True)

    @pl.when(i == nsteps - 1)
    def _():
        scale = _NUM_EXPERTS / (_NUM_TOKENS * _TOP_K * _NUM_TOKENS)
        aux_ref[...] = scale * jnp.sum(cnt_ref[...] * prob_ref[...], keepdims=True)


def kernel(x, W, b):
    n, d = x.shape
    e = W.shape[1]
    grid = (n // _BLOCK,)
    gw, sel, aux = pl.pallas_call(
        _gate_body,
        grid=grid,
        in_specs=[
            pl.BlockSpec((_BLOCK, d), lambda i: (i, 0)),
            pl.BlockSpec((d, e), lambda i: (0, 0)),
            pl.BlockSpec((1, e), lambda i: (0, 0)),
        ],
        out_specs=[
            pl.BlockSpec((_BLOCK, e), lambda i: (i, 0)),
            pl.BlockSpec((_BLOCK, _TOP_K), lambda i: (i, 0)),
            pl.BlockSpec((1, 1), lambda i: (0, 0)),
        ],
        out_shape=[
            jax.ShapeDtypeStruct((n, e), jnp.float32),
            jax.ShapeDtypeStruct((n, _TOP_K), jnp.int32),
            jax.ShapeDtypeStruct((1, 1), jnp.float32),
        ],
        scratch_shapes=[
            pltpu.VMEM((1, e), jnp.float32),
            pltpu.VMEM((1, e), jnp.float32),
        ],
    )(x, W, b.reshape(1, e))
    return gw, sel, aux[0, 0]


# speedup vs baseline: 9.2490x; 1.0024x over previous
import jax
import jax.numpy as jnp
from jax.experimental import pallas as pl
from jax.experimental.pallas import tpu as pltpu

_NUM_TOKENS = 32768
_NUM_EXPERTS = 64
_TOP_K = 2
_BLOCK = 4096


def _gate_body(x_ref, w_ref, b_ref, gw_ref, sel_ref, aux_ref, cnt_ref, prob_ref):
    i = pl.program_id(0)
    nsteps = pl.num_programs(0)
    e = _NUM_EXPERTS

    logits = jnp.dot(x_ref[...], w_ref[...], preferred_element_type=jnp.float32)
    logits = logits + b_ref[...]

    m1 = jnp.max(logits, axis=1, keepdims=True)
    sel1 = logits == m1
    sel1f = jnp.where(sel1, 1.0, 0.0)

    masked = jnp.where(sel1, -jnp.inf, logits)
    m2 = jnp.max(masked, axis=1, keepdims=True)
    sel2f = jnp.where(masked == m2, 1.0, 0.0)

    iota_mat = jax.lax.broadcasted_iota(jnp.int32, (e, e), 0).astype(jnp.float32)
    i1f = jnp.dot(sel1f, iota_mat, preferred_element_type=jnp.float32)
    i2f = jnp.dot(sel2f, iota_mat, preferred_element_type=jnp.float32)

    p_un = jnp.exp(logits - m1)
    mask2f = sel1f + sel2f
    p_sel = p_un * mask2f

    ones_e = jnp.full((e, e), 1.0, dtype=jnp.float32)
    s_full = jnp.dot(p_un, ones_e, preferred_element_type=jnp.float32)
    s_pair = jnp.dot(p_sel, ones_e, preferred_element_type=jnp.float32)

    gw_ref[...] = p_sel / s_pair
    sel_ref[...] = jnp.concatenate(
        [i1f[:, 0:1].astype(jnp.int32), i2f[:, 0:1].astype(jnp.int32)], axis=1)

    q = p_un / s_full

    @pl.when(i == 0)
    def _():
        cnt_ref[...] = jnp.zeros_like(cnt_ref)
        prob_ref[...] = jnp.zeros_like(prob_ref)

    ones_rows = jnp.full((8, _BLOCK), 1.0, dtype=jnp.float32)
    cnt_ref[...] += jnp.dot(ones_rows, mask2f, preferred_element_type=jnp.float32)
    prob_ref[...] += jnp.dot(ones_rows, q, preferred_element_type=jnp.float32)

    @pl.when(i == nsteps - 1)
    def _():
        scale = _NUM_EXPERTS / (_NUM_TOKENS * _TOP_K * _NUM_TOKENS * 64.0)
        cnt1 = jnp.sum(cnt_ref[...], axis=0, keepdims=True)
        prob1 = jnp.sum(prob_ref[...], axis=0, keepdims=True)
        aux_ref[...] = scale * jnp.sum(cnt1 * prob1, keepdims=True)


def kernel(x, W, b):
    n, d = x.shape
    e = W.shape[1]
    grid = (n // _BLOCK,)
    gw, sel, aux = pl.pallas_call(
        _gate_body,
        grid=grid,
        in_specs=[
            pl.BlockSpec((_BLOCK, d), lambda i: (i, 0)),
            pl.BlockSpec((d, e), lambda i: (0, 0)),
            pl.BlockSpec((1, e), lambda i: (0, 0)),
        ],
        out_specs=[
            pl.BlockSpec((_BLOCK, e), lambda i: (i, 0)),
            pl.BlockSpec((_BLOCK, _TOP_K), lambda i: (i, 0)),
            pl.BlockSpec((1, 1), lambda i: (0, 0)),
        ],
        out_shape=[
            jax.ShapeDtypeStruct((n, e), jnp.float32),
            jax.ShapeDtypeStruct((n, _TOP_K), jnp.int32),
            jax.ShapeDtypeStruct((1, 1), jnp.float32),
        ],
        scratch_shapes=[
            pltpu.VMEM((8, e), jnp.float32),
            pltpu.VMEM((8, e), jnp.float32),
        ],
    )(x, W, b.reshape(1, e))
    return gw, sel, aux[0, 0]


# speedup vs baseline: 10.8765x; 1.1760x over previous
import jax
import jax.numpy as jnp
from jax.experimental import pallas as pl
from jax.experimental.pallas import tpu as pltpu

_NUM_TOKENS = 32768
_NUM_EXPERTS = 64
_TOP_K = 2
_BLOCK = 4096


def _gate_body(x_ref, w_ref, b_ref, gw_ref, sel_ref, aux_ref, cnt_ref, prob_ref):
    i = pl.program_id(0)
    nsteps = pl.num_programs(0)
    e = _NUM_EXPERTS

    logits = jnp.dot(x_ref[...], w_ref[...], preferred_element_type=jnp.float32)
    logits = logits + b_ref[...]

    m1 = jnp.max(logits, axis=1, keepdims=True)
    sel1 = logits == m1
    sel1f = jnp.where(sel1, 1.0, 0.0)

    masked = jnp.where(sel1, -jnp.inf, logits)
    m2 = jnp.max(masked, axis=1, keepdims=True)
    sel2f = jnp.where(masked == m2, 1.0, 0.0)

    iota_mat = jax.lax.broadcasted_iota(jnp.int32, (e, e), 0).astype(jnp.float32)
    i1f = jnp.dot(sel1f, iota_mat, preferred_element_type=jnp.float32)
    i2f = jnp.dot(sel2f, iota_mat, preferred_element_type=jnp.float32)

    p_un = jnp.exp(logits - m1)
    mask2f = sel1f + sel2f
    p_sel = p_un * mask2f

    ones_e = jnp.full((e, e), 1.0, dtype=jnp.float32)
    s_full = jnp.dot(p_un, ones_e, preferred_element_type=jnp.float32)
    s_pair = jnp.dot(p_sel, ones_e, preferred_element_type=jnp.float32)

    gw_ref[...] = p_sel / s_pair
    sel_pair = jnp.concatenate(
        [i1f[:, 0:1].astype(jnp.int32), i2f[:, 0:1].astype(jnp.int32)], axis=1)
    sel_ref[...] = sel_pair.T

    q = p_un / s_full

    @pl.when(i == 0)
    def _():
        cnt_ref[...] = jnp.zeros_like(cnt_ref)
        prob_ref[...] = jnp.zeros_like(prob_ref)

    ones_rows = jnp.full((8, _BLOCK), 1.0, dtype=jnp.float32)
    cnt_ref[...] += jnp.dot(ones_rows, mask2f, preferred_element_type=jnp.float32)
    prob_ref[...] += jnp.dot(ones_rows, q, preferred_element_type=jnp.float32)

    @pl.when(i == nsteps - 1)
    def _():
        scale = _NUM_EXPERTS / (_NUM_TOKENS * _TOP_K * _NUM_TOKENS * 64.0)
        cnt1 = jnp.sum(cnt_ref[...], axis=0, keepdims=True)
        prob1 = jnp.sum(prob_ref[...], axis=0, keepdims=True)
        aux_ref[...] = scale * jnp.sum(cnt1 * prob1, keepdims=True)


def kernel(x, W, b):
    n, d = x.shape
    e = W.shape[1]
    grid = (n // _BLOCK,)
    gw, sel_t, aux = pl.pallas_call(
        _gate_body,
        grid=grid,
        in_specs=[
            pl.BlockSpec((_BLOCK, d), lambda i: (i, 0)),
            pl.BlockSpec((d, e), lambda i: (0, 0)),
            pl.BlockSpec((1, e), lambda i: (0, 0)),
        ],
        out_specs=[
            pl.BlockSpec((_BLOCK, e), lambda i: (i, 0)),
            pl.BlockSpec((_TOP_K, _BLOCK), lambda i: (0, i)),
            pl.BlockSpec((1, 1), lambda i: (0, 0)),
        ],
        out_shape=[
            jax.ShapeDtypeStruct((n, e), jnp.float32),
            jax.ShapeDtypeStruct((_TOP_K, n), jnp.int32),
            jax.ShapeDtypeStruct((1, 1), jnp.float32),
        ],
        scratch_shapes=[
            pltpu.VMEM((8, e), jnp.float32),
            pltpu.VMEM((8, e), jnp.float32),
        ],
    )(x, W, b.reshape(1, e))
    return gw, sel_t.T, aux[0, 0]


# speedup vs baseline: 10.9232x; 1.0043x over previous
import jax
import jax.numpy as jnp
from jax.experimental import pallas as pl
from jax.experimental.pallas import tpu as pltpu

_NUM_TOKENS = 32768
_NUM_EXPERTS = 64
_TOP_K = 2
_BLOCK = 4096


def _gate_body(x_ref, w_ref, b_ref, gw_ref, sel_ref, aux_ref, cnt_ref, prob_ref):
    i = pl.program_id(0)
    nsteps = pl.num_programs(0)
    e = _NUM_EXPERTS

    logits = jnp.dot(x_ref[...], w_ref[...], preferred_element_type=jnp.float32)
    logits = logits + b_ref[...]

    m1 = jnp.max(logits, axis=1, keepdims=True)
    sel1 = logits == m1
    sel1f = jnp.where(sel1, 1.0, 0.0)

    masked = jnp.where(sel1, -jnp.inf, logits)
    m2 = jnp.max(masked, axis=1, keepdims=True)
    mask2f = jnp.where(logits >= m2, 1.0, 0.0)

    iota_mat = jax.lax.broadcasted_iota(jnp.int32, (e, e), 0).astype(jnp.float32)
    i1f = jnp.dot(sel1f, iota_mat, preferred_element_type=jnp.float32)
    i12f = jnp.dot(mask2f, iota_mat, preferred_element_type=jnp.float32)

    p_un = jnp.exp(logits - m1)
    p_sel = p_un * mask2f

    ones_e = jnp.full((e, e), 1.0, dtype=jnp.float32)
    s_full = jnp.dot(p_un, ones_e, preferred_element_type=jnp.float32)
    s_pair = jnp.dot(p_sel, ones_e, preferred_element_type=jnp.float32)

    gw_ref[...] = p_sel / s_pair
    i1c = i1f[:, 0:1]
    i2c = i12f[:, 0:1] - i1c
    sel_pair = jnp.concatenate(
        [i1c.astype(jnp.int32), i2c.astype(jnp.int32)], axis=1)
    sel_ref[...] = sel_pair.T

    q = p_un / s_full

    @pl.when(i == 0)
    def _():
        cnt_ref[...] = jnp.zeros_like(cnt_ref)
        prob_ref[...] = jnp.zeros_like(prob_ref)

    ones_rows = jnp.full((8, _BLOCK), 1.0, dtype=jnp.float32)
    cnt_ref[...] += jnp.dot(ones_rows, mask2f, preferred_element_type=jnp.float32)
    prob_ref[...] += jnp.dot(ones_rows, q, preferred_element_type=jnp.float32)

    @pl.when(i == nsteps - 1)
    def _():
        scale = _NUM_EXPERTS / (_NUM_TOKENS * _TOP_K * _NUM_TOKENS * 64.0)
        cnt1 = jnp.sum(cnt_ref[...], axis=0, keepdims=True)
        prob1 = jnp.sum(prob_ref[...], axis=0, keepdims=True)
        aux_ref[...] = scale * jnp.sum(cnt1 * prob1, keepdims=True)


def kernel(x, W, b):
    n, d = x.shape
    e = W.shape[1]
    grid = (n // _BLOCK,)
    gw, sel_t, aux = pl.pallas_call(
        _gate_body,
        grid=grid,
        in_specs=[
            pl.BlockSpec((_BLOCK, d), lambda i: (i, 0)),
            pl.BlockSpec((d, e), lambda i: (0, 0)),
            pl.BlockSpec((1, e), lambda i: (0, 0)),
        ],
        out_specs=[
            pl.BlockSpec((_BLOCK, e), lambda i: (i, 0)),
            pl.BlockSpec((_TOP_K, _BLOCK), lambda i: (0, i)),
            pl.BlockSpec((1, 1), lambda i: (0, 0)),
        ],
        out_shape=[
            jax.ShapeDtypeStruct((n, e), jnp.float32),
            jax.ShapeDtypeStruct((_TOP_K, n), jnp.int32),
            jax.ShapeDtypeStruct((1, 1), jnp.float32),
        ],
        scratch_shapes=[
            pltpu.VMEM((8, e), jnp.float32),
            pltpu.VMEM((8, e), jnp.float32),
        ],
    )(x, W, b.reshape(1, e))
    return gw, sel_t.T, aux[0, 0]
